# Initial kernel scaffold; baseline (speedup 1.0000x reference)
import jax
import jax.numpy as jnp
from jax.experimental import pallas as pl


def kernel(data, x, edge_index, node_batch, char_table, W_l1, b_l1, W_r1, W_l2, b_l2, W_r2, W_out, b_out):
    raise NotImplementedError("write your pallas kernel here")


# speedup vs baseline: 6.0939x; 6.0939x over previous
import functools

import jax
import jax.numpy as jnp
from jax import lax
from jax.experimental import pallas as pl
from jax.experimental.pallas import tpu as pltpu
from jax.experimental.pallas import tpu_sc as plsc

N_NODES = 10000
D = 128
N_EDGES = 320000
NUM_GRAPHS = 1024
NC = 2
NS = 16
NW = NC * NS
E_PER_W = N_EDGES // NW
CHUNK = 80
KIN = 5
KOUT = E_PER_W // (CHUNK * KIN)
N_PAD = 10240
ROWS_PER_S = N_PAD // NS
G_PER_W = NUM_GRAPHS // NW
PCH = 32

_mesh = plsc.VectorSubcoreMesh(core_axis_name="c", subcore_axis_name="s")


def _zero_vmem(ref, nrows, width):
  def row(i, carry):
    for j in range(width // 16):
      ref[i, pl.ds(16 * j, 16)] = jnp.zeros((16,), jnp.float32)
    return carry
  lax.fori_loop(0, nrows, row, 0)


def _edge_body(feat, srcr, dstr, agg_out, src_v, dst_v, rows_v, agg_sh, sem):
  cid = lax.axis_index("c")
  sid = lax.axis_index("s")
  wid = sid * NC + cid

  _zero_vmem(rows_v, CHUNK, D)
  for c in range(ROWS_PER_S // CHUNK):
    off = sid * ROWS_PER_S + c * CHUNK
    pltpu.sync_copy(rows_v, agg_sh.at[pl.ds(off, CHUNK)])
  plsc.subcore_barrier()

  def outer(k, carry):
    pltpu.sync_copy(srcr.at[wid, k], src_v)
    pltpu.sync_copy(dstr.at[wid, k], dst_v)
    for j in range(KIN):
      pltpu.async_copy(feat.at[src_v.at[j]], rows_v, sem).wait()
      pltpu.sync_copy(rows_v, agg_sh.at[dst_v.at[j]], add=True)
    return carry
  lax.fori_loop(0, KOUT, outer, 0)

  plsc.subcore_barrier()
  for c in range(ROWS_PER_S // CHUNK):
    off = sid * ROWS_PER_S + c * CHUNK
    pltpu.sync_copy(agg_sh.at[pl.ds(off, CHUNK)],
                    agg_out.at[cid, pl.ds(off, CHUNK)])


_edge_kernel = functools.partial(
    pl.kernel,
    out_type=jax.ShapeDtypeStruct((NC, N_PAD, D), jnp.float32),
    mesh=_mesh,
    scratch_types=[
        pltpu.VMEM((KIN, CHUNK), jnp.int32),
        pltpu.VMEM((KIN, CHUNK), jnp.int32),
        pltpu.VMEM((CHUNK, D), jnp.float32),
        pltpu.VMEM_SHARED((N_PAD, D), jnp.float32),
        pltpu.SemaphoreType.DMA,
    ],
    compiler_params=pltpu.CompilerParams(needs_layout_passes=False),
)(_edge_body)


def _deg_body(dstr, deg_out, dst_v, hist_v, sem):
  cid = lax.axis_index("c")
  sid = lax.axis_index("s")
  wid = sid * NC + cid

  _zero_vmem(hist_v, N_PAD // D, D)
  ones16 = jnp.full((16,), 1.0, jnp.float32)

  def outer(k, carry):
    pltpu.sync_copy(dstr.at[wid, k], dst_v)
    for j in range(KIN):
      for t in range(CHUNK // 16):
        idx = dst_v[j, pl.ds(16 * t, 16)]
        plsc.addupdate_scatter(hist_v, [idx >> 7, idx & 127], ones16)
    return carry
  lax.fori_loop(0, KOUT, outer, 0)

  pltpu.sync_copy(hist_v, deg_out.at[wid])


_deg_kernel = functools.partial(
    pl.kernel,
    out_type=jax.ShapeDtypeStruct((NW, N_PAD // D, D), jnp.float32),
    mesh=_mesh,
    scratch_types=[
        pltpu.VMEM((KIN, CHUNK), jnp.int32),
        pltpu.VMEM((N_PAD // D, D), jnp.float32),
        pltpu.SemaphoreType.DMA,
    ],
    compiler_params=pltpu.CompilerParams(needs_layout_passes=False),
)(_deg_body)


def _pool_body(h2r, nbp, idsr, ctab, pool_out, char_out,
               pool_v, rows_v, nbc_v, probe_v, cidx_v, crow_v, sem):
  cid = lax.axis_index("c")
  sid = lax.axis_index("s")
  wid = sid * NC + cid
  g0 = wid * G_PER_W

  pltpu.sync_copy(idsr, cidx_v)
  pltpu.async_copy(ctab.at[cidx_v.at[pl.ds(g0, G_PER_W)]], crow_v, sem).wait()
  pltpu.sync_copy(crow_v, char_out.at[pl.ds(g0, G_PER_W)])

  _zero_vmem(pool_v, G_PER_W, D)

  def search(target):
    def body(_, s):
      lo, hi = s
      mid = (lo + hi) // 2
      pltpu.sync_copy(nbp.at[pl.ds(mid * 8, 16)], probe_v.at[pl.ds(0, 16)])
      v = probe_v[pl.ds(0, 16)][0]
      lo2 = jnp.where(v < target, mid + 1, lo)
      hi2 = jnp.where(v < target, hi, mid)
      return (lo2, hi2)
    p, _ = lax.fori_loop(0, 11, body, (0, N_NODES // 8))
    base = jnp.maximum(p - 1, 0) * 8
    pltpu.sync_copy(nbp.at[pl.ds(base, 16)], probe_v.at[pl.ds(0, 16)])
    cnt = jnp.int32(0)
    for i in range(8):
      vi = probe_v[pl.ds(i, 16)][0]
      cnt = cnt + jnp.where(vi < target, 1, 0)
    return base + jnp.where(p > 0, cnt, 0)

  lo = search(g0)
  hi = search(g0 + G_PER_W)

  la = (lo // 8) * 8
  nchunks = (hi - la + PCH - 1) // PCH

  def ck(k, carry):
    n0 = la + k * PCH
    n0c = jnp.minimum(n0, N_NODES - PCH)
    pltpu.sync_copy(h2r.at[pl.ds(n0c, PCH)], rows_v)
    pltpu.sync_copy(nbp.at[pl.ds(n0c, PCH)], nbc_v.at[pl.ds(0, PCH)])

    def rb(r, c2):
      n = n0c + r
      ok = jnp.logical_and(n >= lo, n < hi)
      @pl.when(ok)
      def _():
        tgt = nbc_v[pl.ds(r, 16)][0] - g0
        for j in range(D // 16):
          sl = pl.ds(16 * j, 16)
          pool_v[tgt, sl] = jnp.maximum(pool_v[tgt, sl], rows_v[r, sl])
      return c2
    lax.fori_loop(0, PCH, rb, 0)
    return carry
  lax.fori_loop(0, nchunks, ck, 0)

  pltpu.sync_copy(pool_v, pool_out.at[pl.ds(g0, G_PER_W)])


_pool_kernel = functools.partial(
    pl.kernel,
    out_type=(
        jax.ShapeDtypeStruct((NUM_GRAPHS, D), jnp.float32),
        jax.ShapeDtypeStruct((NUM_GRAPHS, D), jnp.float32),
    ),
    mesh=_mesh,
    scratch_types=[
        pltpu.VMEM((G_PER_W, D), jnp.float32),
        pltpu.VMEM((PCH, D), jnp.float32),
        pltpu.VMEM((PCH + 16,), jnp.int32),
        pltpu.VMEM((32,), jnp.int32),
        pltpu.VMEM((NUM_GRAPHS,), jnp.int32),
        pltpu.VMEM((G_PER_W, D), jnp.float32),
        pltpu.SemaphoreType.DMA,
    ],
    compiler_params=pltpu.CompilerParams(needs_layout_passes=False),
)(_pool_body)


def _dense_body(agg_ref, deg_ref, x_ref, wl_ref, b_ref, wr_ref, o_ref):
  agg = agg_ref[0] + agg_ref[1]
  bn = agg.shape[0]
  hist = jnp.sum(deg_ref[...], axis=0)
  hrep = jnp.broadcast_to(hist[:, None, :], (bn // D, D, D)).reshape(bn, D)
  rows = lax.broadcasted_iota(jnp.int32, (bn, D), 0)
  cols = lax.broadcasted_iota(jnp.int32, (bn, D), 1)
  deg = jnp.sum(jnp.where((rows % D) == cols, hrep, 0.0), axis=1,
                keepdims=True)
  mean = agg / jnp.maximum(deg, 1.0)
  h = lax.dot_general(mean, wl_ref[...], (((1,), (1,)), ((), ())),
                      preferred_element_type=jnp.float32)
  h = h + b_ref[...][None, :]
  h = h + lax.dot_general(x_ref[...], wr_ref[...], (((1,), (1,)), ((), ())),
                          preferred_element_type=jnp.float32)
  o_ref[...] = jnp.maximum(h, 0.0)


def _dense(agg, deg, x, W_l, b_l, W_r):
  bn = 1024
  grid = (N_PAD // bn,)
  return pl.pallas_call(
      _dense_body,
      grid=grid,
      in_specs=[
          pl.BlockSpec((NC, bn, D), lambda i: (0, i, 0)),
          pl.BlockSpec((NW, bn // D, D), lambda i: (0, i, 0)),
          pl.BlockSpec((bn, D), lambda i: (i, 0)),
          pl.BlockSpec((D, D), lambda i: (0, 0)),
          pl.BlockSpec((D,), lambda i: (0,)),
          pl.BlockSpec((D, D), lambda i: (0, 0)),
      ],
      out_specs=pl.BlockSpec((bn, D), lambda i: (i, 0)),
      out_shape=jax.ShapeDtypeStruct((N_PAD, D), jnp.float32),
  )(agg, deg, x, W_l, b_l, W_r)


def _final_body(ch_ref, pool_ref, w_ref, b_ref, o_ref):
  o_ref[:, 0:D] = ch_ref[...]
  gf = lax.dot_general(pool_ref[...], w_ref[...], (((1,), (1,)), ((), ())),
                       preferred_element_type=jnp.float32)
  o_ref[:, D:2 * D] = gf + b_ref[...][None, :]


def _final(char_e, pooled, W_out, b_out):
  return pl.pallas_call(
      _final_body,
      out_shape=jax.ShapeDtypeStruct((NUM_GRAPHS, 2 * D), jnp.float32),
  )(char_e, pooled, W_out, b_out)


def kernel(data, x, edge_index, node_batch, char_table,
           W_l1, b_l1, W_r1, W_l2, b_l2, W_r2, W_out, b_out):
  ids = data.reshape(NUM_GRAPHS)
  srcr = edge_index[0].reshape(NW, KOUT, KIN, CHUNK)
  dstr = edge_index[1].reshape(NW, KOUT, KIN, CHUNK)
  x_pad = jnp.pad(x, ((0, N_PAD - N_NODES), (0, 0)))
  nbp = jnp.pad(node_batch, (0, 16), constant_values=NUM_GRAPHS)

  agg1 = _edge_kernel(x_pad, srcr, dstr)
  deg = _deg_kernel(dstr)
  h1 = _dense(agg1, deg, x_pad, W_l1, b_l1, W_r1)
  agg2 = _edge_kernel(h1, srcr, dstr)
  h2 = _dense(agg2, deg, h1, W_l2, b_l2, W_r2)

  pooled, char_e = _pool_kernel(h2, nbp, ids, char_table)
  emb = _final(char_e, pooled, W_out, b_out)
  return emb.reshape(32, 32, 2 * D)


# speedup vs baseline: 8.3001x; 1.3620x over previous
import functools

import jax
import jax.numpy as jnp
from jax import lax
from jax.experimental import pallas as pl
from jax.experimental.pallas import tpu as pltpu
from jax.experimental.pallas import tpu_sc as plsc

N_NODES = 10000
D = 128
N_EDGES = 320000
NUM_GRAPHS = 1024
NC = 2
NS = 16
NW = NC * NS
E_PER_W = N_EDGES // NW
CHUNK = 80
KIN = 5
KOUT = E_PER_W // (CHUNK * KIN)
ECH = 125
EKIN = 4
EKOUT = E_PER_W // (ECH * EKIN)
N_PAD = 10240
ROWS_PER_S = N_PAD // NS
G_PER_W = NUM_GRAPHS // NW
PCH = 32

_mesh = plsc.VectorSubcoreMesh(core_axis_name="c", subcore_axis_name="s")


def _zero_vmem(ref, nrows, width):
  def row(i, carry):
    for j in range(width // 16):
      ref[i, pl.ds(16 * j, 16)] = jnp.zeros((16,), jnp.float32)
    return carry
  lax.fori_loop(0, nrows, row, 0)


def _edge_body(feat, srcr, dstr, agg_out, src_v, dst_v, rows_a, rows_b,
               agg_sh, sem_a, sem_b):
  cid = lax.axis_index("c")
  sid = lax.axis_index("s")
  wid = sid * NC + cid

  _zero_vmem(rows_a, 80, D)
  for c in range(ROWS_PER_S // 80):
    off = sid * ROWS_PER_S + c * 80
    pltpu.sync_copy(rows_a.at[pl.ds(0, 80)], agg_sh.at[pl.ds(off, 80)])
  plsc.subcore_barrier()

  def outer(k, carry):
    pltpu.sync_copy(srcr.at[wid, k], src_v)
    pltpu.sync_copy(dstr.at[wid, k], dst_v)
    descs = [None] * EKIN
    descs[0] = pltpu.async_copy(feat.at[src_v.at[0]], rows_a, sem_a)
    for j in range(EKIN):
      cur = rows_a if j % 2 == 0 else rows_b
      if j + 1 < EKIN:
        nxt = rows_b if j % 2 == 0 else rows_a
        nsem = sem_b if j % 2 == 0 else sem_a
        descs[j + 1] = pltpu.async_copy(feat.at[src_v.at[j + 1]], nxt, nsem)
      descs[j].wait()
      pltpu.sync_copy(cur, agg_sh.at[dst_v.at[j]], add=True)
    return carry
  lax.fori_loop(0, EKOUT, outer, 0)

  plsc.subcore_barrier()
  for c in range(ROWS_PER_S // 80):
    off = sid * ROWS_PER_S + c * 80
    pltpu.sync_copy(agg_sh.at[pl.ds(off, 80)],
                    agg_out.at[cid, pl.ds(off, 80)])


_edge_kernel = functools.partial(
    pl.kernel,
    out_type=jax.ShapeDtypeStruct((NC, N_PAD, D), jnp.float32),
    mesh=_mesh,
    scratch_types=[
        pltpu.VMEM((EKIN, ECH), jnp.int32),
        pltpu.VMEM((EKIN, ECH), jnp.int32),
        pltpu.VMEM((ECH, D), jnp.float32),
        pltpu.VMEM((ECH, D), jnp.float32),
        pltpu.VMEM_SHARED((N_PAD, D), jnp.float32),
        pltpu.SemaphoreType.DMA,
        pltpu.SemaphoreType.DMA,
    ],
    compiler_params=pltpu.CompilerParams(needs_layout_passes=False),
)(_edge_body)


def _deg_body(dstr, deg_out, dst_v, hist_v, sem):
  cid = lax.axis_index("c")
  sid = lax.axis_index("s")
  wid = sid * NC + cid

  _zero_vmem(hist_v, N_PAD // D, D)
  ones16 = jnp.full((16,), 1.0, jnp.float32)

  def outer(k, carry):
    pltpu.sync_copy(dstr.at[wid, k], dst_v)
    for j in range(KIN):
      for t in range(CHUNK // 16):
        idx = dst_v[j, pl.ds(16 * t, 16)]
        plsc.addupdate_scatter(hist_v, [idx >> 7, idx & 127], ones16)
    return carry
  lax.fori_loop(0, KOUT, outer, 0)

  pltpu.sync_copy(hist_v, deg_out.at[wid])


_deg_kernel = functools.partial(
    pl.kernel,
    out_type=jax.ShapeDtypeStruct((NW, N_PAD // D, D), jnp.float32),
    mesh=_mesh,
    scratch_types=[
        pltpu.VMEM((KIN, CHUNK), jnp.int32),
        pltpu.VMEM((N_PAD // D, D), jnp.float32),
        pltpu.SemaphoreType.DMA,
    ],
    compiler_params=pltpu.CompilerParams(needs_layout_passes=False),
)(_deg_body)


def _pool_body(h2r, nbp, idsr, ctab, pool_out, char_out,
               pool_v, rows_v, nbc_v, probe_v, cidx_v, crow_v, sem):
  cid = lax.axis_index("c")
  sid = lax.axis_index("s")
  wid = sid * NC + cid
  g0 = wid * G_PER_W

  pltpu.sync_copy(idsr, cidx_v)
  pltpu.async_copy(ctab.at[cidx_v.at[pl.ds(g0, G_PER_W)]], crow_v, sem).wait()
  pltpu.sync_copy(crow_v, char_out.at[pl.ds(g0, G_PER_W)])

  _zero_vmem(pool_v, G_PER_W, D)

  def search(target):
    def body(_, s):
      lo, hi = s
      mid = (lo + hi) // 2
      pltpu.sync_copy(nbp.at[pl.ds(mid * 8, 16)], probe_v.at[pl.ds(0, 16)])
      v = probe_v[pl.ds(0, 16)][0]
      lo2 = jnp.where(v < target, mid + 1, lo)
      hi2 = jnp.where(v < target, hi, mid)
      return (lo2, hi2)
    p, _ = lax.fori_loop(0, 11, body, (0, N_NODES // 8))
    base = jnp.maximum(p - 1, 0) * 8
    pltpu.sync_copy(nbp.at[pl.ds(base, 16)], probe_v.at[pl.ds(0, 16)])
    cnt = jnp.int32(0)
    for i in range(8):
      vi = probe_v[pl.ds(i, 16)][0]
      cnt = cnt + jnp.where(vi < target, 1, 0)
    return base + jnp.where(p > 0, cnt, 0)

  lo = search(g0)
  hi = search(g0 + G_PER_W)

  la = (lo // 8) * 8
  nchunks = (hi - la + PCH - 1) // PCH

  def ck(k, carry):
    n0 = la + k * PCH
    n0c = jnp.minimum(n0, N_NODES - PCH)
    pltpu.sync_copy(h2r.at[pl.ds(n0c, PCH)], rows_v)
    pltpu.sync_copy(nbp.at[pl.ds(n0c, PCH)], nbc_v.at[pl.ds(0, PCH)])

    def rb(r, c2):
      n = n0c + r
      ok = jnp.logical_and(n >= lo, n < hi)
      @pl.when(ok)
      def _():
        tgt = nbc_v[pl.ds(r, 16)][0] - g0
        for j in range(D // 16):
          sl = pl.ds(16 * j, 16)
          pool_v[tgt, sl] = jnp.maximum(pool_v[tgt, sl], rows_v[r, sl])
      return c2
    lax.fori_loop(0, PCH, rb, 0)
    return carry
  lax.fori_loop(0, nchunks, ck, 0)

  pltpu.sync_copy(pool_v, pool_out.at[pl.ds(g0, G_PER_W)])


_pool_kernel = functools.partial(
    pl.kernel,
    out_type=(
        jax.ShapeDtypeStruct((NUM_GRAPHS, D), jnp.float32),
        jax.ShapeDtypeStruct((NUM_GRAPHS, D), jnp.float32),
    ),
    mesh=_mesh,
    scratch_types=[
        pltpu.VMEM((G_PER_W, D), jnp.float32),
        pltpu.VMEM((PCH, D), jnp.float32),
        pltpu.VMEM((PCH + 16,), jnp.int32),
        pltpu.VMEM((32,), jnp.int32),
        pltpu.VMEM((NUM_GRAPHS,), jnp.int32),
        pltpu.VMEM((G_PER_W, D), jnp.float32),
        pltpu.SemaphoreType.DMA,
    ],
    compiler_params=pltpu.CompilerParams(needs_layout_passes=False),
)(_pool_body)


def _dense_body(agg_ref, deg_ref, x_ref, wl_ref, b_ref, wr_ref, o_ref):
  agg = agg_ref[0] + agg_ref[1]
  bn = agg.shape[0]
  hist = jnp.sum(deg_ref[...], axis=0)
  hrep = jnp.broadcast_to(hist[:, None, :], (bn // D, D, D)).reshape(bn, D)
  rows = lax.broadcasted_iota(jnp.int32, (bn, D), 0)
  cols = lax.broadcasted_iota(jnp.int32, (bn, D), 1)
  deg = jnp.sum(jnp.where((rows % D) == cols, hrep, 0.0), axis=1,
                keepdims=True)
  mean = agg / jnp.maximum(deg, 1.0)
  h = lax.dot_general(mean, wl_ref[...], (((1,), (1,)), ((), ())),
                      preferred_element_type=jnp.float32)
  h = h + b_ref[...][None, :]
  h = h + lax.dot_general(x_ref[...], wr_ref[...], (((1,), (1,)), ((), ())),
                          preferred_element_type=jnp.float32)
  o_ref[...] = jnp.maximum(h, 0.0)


def _dense(agg, deg, x, W_l, b_l, W_r):
  bn = 1024
  grid = (N_PAD // bn,)
  return pl.pallas_call(
      _dense_body,
      grid=grid,
      in_specs=[
          pl.BlockSpec((NC, bn, D), lambda i: (0, i, 0)),
          pl.BlockSpec((NW, bn // D, D), lambda i: (0, i, 0)),
          pl.BlockSpec((bn, D), lambda i: (i, 0)),
          pl.BlockSpec((D, D), lambda i: (0, 0)),
          pl.BlockSpec((D,), lambda i: (0,)),
          pl.BlockSpec((D, D), lambda i: (0, 0)),
      ],
      out_specs=pl.BlockSpec((bn, D), lambda i: (i, 0)),
      out_shape=jax.ShapeDtypeStruct((N_PAD, D), jnp.float32),
  )(agg, deg, x, W_l, b_l, W_r)


def _final_body(ch_ref, pool_ref, w_ref, b_ref, o_ref):
  o_ref[:, 0:D] = ch_ref[...]
  gf = lax.dot_general(pool_ref[...], w_ref[...], (((1,), (1,)), ((), ())),
                       preferred_element_type=jnp.float32)
  o_ref[:, D:2 * D] = gf + b_ref[...][None, :]


def _final(char_e, pooled, W_out, b_out):
  return pl.pallas_call(
      _final_body,
      out_shape=jax.ShapeDtypeStruct((NUM_GRAPHS, 2 * D), jnp.float32),
  )(char_e, pooled, W_out, b_out)


def kernel(data, x, edge_index, node_batch, char_table,
           W_l1, b_l1, W_r1, W_l2, b_l2, W_r2, W_out, b_out):
  ids = data.reshape(NUM_GRAPHS)
  srcr = edge_index[0].reshape(NW, EKOUT, EKIN, ECH)
  dstr = edge_index[1].reshape(NW, EKOUT, EKIN, ECH)
  dstr_deg = edge_index[1].reshape(NW, KOUT, KIN, CHUNK)
  x_pad = jnp.pad(x, ((0, N_PAD - N_NODES), (0, 0)))
  nbp = jnp.pad(node_batch, (0, 16), constant_values=NUM_GRAPHS)

  agg1 = _edge_kernel(x_pad, srcr, dstr)
  deg = _deg_kernel(dstr_deg)
  h1 = _dense(agg1, deg, x_pad, W_l1, b_l1, W_r1)
  agg2 = _edge_kernel(h1, srcr, dstr)
  h2 = _dense(agg2, deg, h1, W_l2, b_l2, W_r2)

  pooled, char_e = _pool_kernel(h2, nbp, ids, char_table)
  emb = _final(char_e, pooled, W_out, b_out)
  return emb.reshape(32, 32, 2 * D)


# speedup vs baseline: 9.2621x; 1.1159x over previous
import functools

import jax
import jax.numpy as jnp
from jax import lax
from jax.experimental import pallas as pl
from jax.experimental.pallas import tpu as pltpu
from jax.experimental.pallas import tpu_sc as plsc

N_NODES = 10000
D = 128
N_EDGES = 320000
NUM_GRAPHS = 1024
NC = 2
NS = 16
NW = NC * NS
E_PER_W = N_EDGES // NW
CHUNK = 80
KIN = 5
KOUT = E_PER_W // (CHUNK * KIN)
ECH = 125
EKIN = 4
EKOUT = E_PER_W // (ECH * EKIN)
N_PAD = 10240
ROWS_PER_S = N_PAD // NS
G_PER_W = NUM_GRAPHS // NW
PCH = 32

_mesh = plsc.VectorSubcoreMesh(core_axis_name="c", subcore_axis_name="s")


def _zero_vmem(ref, nrows, width):
  def row(i, carry):
    for j in range(width // 16):
      ref[i, pl.ds(16 * j, 16)] = jnp.zeros((16,), jnp.float32)
    return carry
  lax.fori_loop(0, nrows, row, 0)


def _edge_body(feat, srcr, dstr, agg_out, src_a, dst_a, src_b, dst_b,
               rows_a, rows_b, agg_sh, sem_a, sem_b, sem_ia, sem_ib):
  cid = lax.axis_index("c")
  sid = lax.axis_index("s")
  wid = sid * NC + cid
  M = EKOUT // 2

  _zero_vmem(rows_a, 80, D)
  for c in range(ROWS_PER_S // 80):
    off = sid * ROWS_PER_S + c * 80
    pltpu.sync_copy(rows_a.at[pl.ds(0, 80)], agg_sh.at[pl.ds(off, 80)])
  plsc.subcore_barrier()

  pltpu.sync_copy(srcr.at[wid, 0], src_a)
  pltpu.sync_copy(dstr.at[wid, 0], dst_a)
  pltpu.async_copy(feat.at[src_a.at[0]], rows_a, sem_a)

  def outer(m, carry):
    dib0 = pltpu.async_copy(srcr.at[wid, 2 * m + 1], src_b, sem_ib)
    dib1 = pltpu.async_copy(dstr.at[wid, 2 * m + 1], dst_b, sem_ib)
    for j in range(EKIN):
      cur, csem = (rows_a, sem_a) if j % 2 == 0 else (rows_b, sem_b)
      nxt, nsem = (rows_b, sem_b) if j % 2 == 0 else (rows_a, sem_a)
      pltpu.make_async_copy(feat.at[src_a.at[j]], cur, csem).wait()
      if j + 1 < EKIN:
        pltpu.async_copy(feat.at[src_a.at[j + 1]], nxt, nsem)
      else:
        dib0.wait()
        dib1.wait()
        pltpu.async_copy(feat.at[src_b.at[0]], nxt, nsem)
      pltpu.sync_copy(cur, agg_sh.at[dst_a.at[j]], add=True)

    @pl.when(m < M - 1)
    def _():
      pltpu.async_copy(srcr.at[wid, 2 * m + 2], src_a, sem_ia)
      pltpu.async_copy(dstr.at[wid, 2 * m + 2], dst_a, sem_ia)

    for j in range(EKIN):
      cur, csem = (rows_a, sem_a) if j % 2 == 0 else (rows_b, sem_b)
      nxt, nsem = (rows_b, sem_b) if j % 2 == 0 else (rows_a, sem_a)
      pltpu.make_async_copy(feat.at[src_b.at[j]], cur, csem).wait()
      if j + 1 < EKIN:
        pltpu.async_copy(feat.at[src_b.at[j + 1]], nxt, nsem)
      else:
        @pl.when(m < M - 1)
        def _():
          pltpu.make_async_copy(srcr.at[wid, 0], src_a, sem_ia).wait()
          pltpu.make_async_copy(dstr.at[wid, 0], dst_a, sem_ia).wait()
          pltpu.async_copy(feat.at[src_a.at[0]], nxt, nsem)
      pltpu.sync_copy(cur, agg_sh.at[dst_b.at[j]], add=True)
    return carry
  lax.fori_loop(0, M, outer, 0)

  plsc.subcore_barrier()
  for c in range(ROWS_PER_S // 80):
    off = sid * ROWS_PER_S + c * 80
    pltpu.sync_copy(agg_sh.at[pl.ds(off, 80)],
                    agg_out.at[cid, pl.ds(off, 80)])


_edge_kernel = functools.partial(
    pl.kernel,
    out_type=jax.ShapeDtypeStruct((NC, N_PAD, D), jnp.float32),
    mesh=_mesh,
    scratch_types=[
        pltpu.VMEM((EKIN, ECH), jnp.int32),
        pltpu.VMEM((EKIN, ECH), jnp.int32),
        pltpu.VMEM((EKIN, ECH), jnp.int32),
        pltpu.VMEM((EKIN, ECH), jnp.int32),
        pltpu.VMEM((ECH, D), jnp.float32),
        pltpu.VMEM((ECH, D), jnp.float32),
        pltpu.VMEM_SHARED((N_PAD, D), jnp.float32),
        pltpu.SemaphoreType.DMA,
        pltpu.SemaphoreType.DMA,
        pltpu.SemaphoreType.DMA,
        pltpu.SemaphoreType.DMA,
    ],
    compiler_params=pltpu.CompilerParams(needs_layout_passes=False),
)(_edge_body)


def _deg_body(dstr, deg_out, dst_v, hist_v, sem):
  cid = lax.axis_index("c")
  sid = lax.axis_index("s")
  wid = sid * NC + cid

  _zero_vmem(hist_v, N_PAD // D, D)
  ones16 = jnp.full((16,), 1.0, jnp.float32)

  def outer(k, carry):
    pltpu.sync_copy(dstr.at[wid, k], dst_v)
    for j in range(KIN):
      for t in range(CHUNK // 16):
        idx = dst_v[j, pl.ds(16 * t, 16)]
        plsc.addupdate_scatter(hist_v, [idx >> 7, idx & 127], ones16)
    return carry
  lax.fori_loop(0, KOUT, outer, 0)

  pltpu.sync_copy(hist_v, deg_out.at[wid])


_deg_kernel = functools.partial(
    pl.kernel,
    out_type=jax.ShapeDtypeStruct((NW, N_PAD // D, D), jnp.float32),
    mesh=_mesh,
    scratch_types=[
        pltpu.VMEM((KIN, CHUNK), jnp.int32),
        pltpu.VMEM((N_PAD // D, D), jnp.float32),
        pltpu.SemaphoreType.DMA,
    ],
    compiler_params=pltpu.CompilerParams(needs_layout_passes=False),
)(_deg_body)


def _pool_body(h2r, nbp, idsr, ctab, pool_out, char_out,
               pool_v, rows_v, nbc_v, probe_v, cidx_v, crow_v, sem):
  cid = lax.axis_index("c")
  sid = lax.axis_index("s")
  wid = sid * NC + cid
  g0 = wid * G_PER_W

  pltpu.sync_copy(idsr, cidx_v)
  pltpu.async_copy(ctab.at[cidx_v.at[pl.ds(g0, G_PER_W)]], crow_v, sem).wait()
  pltpu.sync_copy(crow_v, char_out.at[pl.ds(g0, G_PER_W)])

  _zero_vmem(pool_v, G_PER_W, D)

  def search(target):
    def body(_, s):
      lo, hi = s
      mid = (lo + hi) // 2
      pltpu.sync_copy(nbp.at[pl.ds(mid * 8, 16)], probe_v.at[pl.ds(0, 16)])
      v = probe_v[pl.ds(0, 16)][0]
      lo2 = jnp.where(v < target, mid + 1, lo)
      hi2 = jnp.where(v < target, hi, mid)
      return (lo2, hi2)
    p, _ = lax.fori_loop(0, 11, body, (0, N_NODES // 8))
    base = jnp.maximum(p - 1, 0) * 8
    pltpu.sync_copy(nbp.at[pl.ds(base, 16)], probe_v.at[pl.ds(0, 16)])
    cnt = jnp.int32(0)
    for i in range(8):
      vi = probe_v[pl.ds(i, 16)][0]
      cnt = cnt + jnp.where(vi < target, 1, 0)
    return base + jnp.where(p > 0, cnt, 0)

  lo = search(g0)
  hi = search(g0 + G_PER_W)

  la = (lo // 8) * 8
  nchunks = (hi - la + PCH - 1) // PCH

  def ck(k, carry):
    n0 = la + k * PCH
    n0c = jnp.minimum(n0, N_NODES - PCH)
    pltpu.sync_copy(h2r.at[pl.ds(n0c, PCH)], rows_v)
    pltpu.sync_copy(nbp.at[pl.ds(n0c, PCH)], nbc_v.at[pl.ds(0, PCH)])

    def rb(r, c2):
      n = n0c + r
      ok = jnp.logical_and(n >= lo, n < hi)
      @pl.when(ok)
      def _():
        tgt = nbc_v[pl.ds(r, 16)][0] - g0
        for j in range(D // 16):
          sl = pl.ds(16 * j, 16)
          pool_v[tgt, sl] = jnp.maximum(pool_v[tgt, sl], rows_v[r, sl])
      return c2
    lax.fori_loop(0, PCH, rb, 0)
    return carry
  lax.fori_loop(0, nchunks, ck, 0)

  pltpu.sync_copy(pool_v, pool_out.at[pl.ds(g0, G_PER_W)])


_pool_kernel = functools.partial(
    pl.kernel,
    out_type=(
        jax.ShapeDtypeStruct((NUM_GRAPHS, D), jnp.float32),
        jax.ShapeDtypeStruct((NUM_GRAPHS, D), jnp.float32),
    ),
    mesh=_mesh,
    scratch_types=[
        pltpu.VMEM((G_PER_W, D), jnp.float32),
        pltpu.VMEM((PCH, D), jnp.float32),
        pltpu.VMEM((PCH + 16,), jnp.int32),
        pltpu.VMEM((32,), jnp.int32),
        pltpu.VMEM((NUM_GRAPHS,), jnp.int32),
        pltpu.VMEM((G_PER_W, D), jnp.float32),
        pltpu.SemaphoreType.DMA,
    ],
    compiler_params=pltpu.CompilerParams(needs_layout_passes=False),
)(_pool_body)


def _dense_body(agg_ref, deg_ref, x_ref, wl_ref, b_ref, wr_ref, o_ref):
  agg = agg_ref[0] + agg_ref[1]
  bn = agg.shape[0]
  hist = jnp.sum(deg_ref[...], axis=0)
  hrep = jnp.broadcast_to(hist[:, None, :], (bn // D, D, D)).reshape(bn, D)
  rows = lax.broadcasted_iota(jnp.int32, (bn, D), 0)
  cols = lax.broadcasted_iota(jnp.int32, (bn, D), 1)
  deg = jnp.sum(jnp.where((rows % D) == cols, hrep, 0.0), axis=1,
                keepdims=True)
  mean = agg / jnp.maximum(deg, 1.0)
  h = lax.dot_general(mean, wl_ref[...], (((1,), (1,)), ((), ())),
                      preferred_element_type=jnp.float32)
  h = h + b_ref[...][None, :]
  h = h + lax.dot_general(x_ref[...], wr_ref[...], (((1,), (1,)), ((), ())),
                          preferred_element_type=jnp.float32)
  o_ref[...] = jnp.maximum(h, 0.0)


def _dense(agg, deg, x, W_l, b_l, W_r):
  bn = 1024
  grid = (N_PAD // bn,)
  return pl.pallas_call(
      _dense_body,
      grid=grid,
      in_specs=[
          pl.BlockSpec((NC, bn, D), lambda i: (0, i, 0)),
          pl.BlockSpec((NW, bn // D, D), lambda i: (0, i, 0)),
          pl.BlockSpec((bn, D), lambda i: (i, 0)),
          pl.BlockSpec((D, D), lambda i: (0, 0)),
          pl.BlockSpec((D,), lambda i: (0,)),
          pl.BlockSpec((D, D), lambda i: (0, 0)),
      ],
      out_specs=pl.BlockSpec((bn, D), lambda i: (i, 0)),
      out_shape=jax.ShapeDtypeStruct((N_PAD, D), jnp.float32),
  )(agg, deg, x, W_l, b_l, W_r)


def _final_body(ch_ref, pool_ref, w_ref, b_ref, o_ref):
  o_ref[:, 0:D] = ch_ref[...]
  gf = lax.dot_general(pool_ref[...], w_ref[...], (((1,), (1,)), ((), ())),
                       preferred_element_type=jnp.float32)
  o_ref[:, D:2 * D] = gf + b_ref[...][None, :]


def _final(char_e, pooled, W_out, b_out):
  return pl.pallas_call(
      _final_body,
      out_shape=jax.ShapeDtypeStruct((NUM_GRAPHS, 2 * D), jnp.float32),
  )(char_e, pooled, W_out, b_out)


def kernel(data, x, edge_index, node_batch, char_table,
           W_l1, b_l1, W_r1, W_l2, b_l2, W_r2, W_out, b_out):
  ids = data.reshape(NUM_GRAPHS)
  srcr = edge_index[0].reshape(NW, EKOUT, EKIN, ECH)
  dstr = edge_index[1].reshape(NW, EKOUT, EKIN, ECH)
  dstr_deg = edge_index[1].reshape(NW, KOUT, KIN, CHUNK)
  x_pad = jnp.pad(x, ((0, N_PAD - N_NODES), (0, 0)))
  nbp = jnp.pad(node_batch, (0, 16), constant_values=NUM_GRAPHS)

  agg1 = _edge_kernel(x_pad, srcr, dstr)
  deg = _deg_kernel(dstr_deg)
  h1 = _dense(agg1, deg, x_pad, W_l1, b_l1, W_r1)
  agg2 = _edge_kernel(h1, srcr, dstr)
  h2 = _dense(agg2, deg, h1, W_l2, b_l2, W_r2)

  pooled, char_e = _pool_kernel(h2, nbp, ids, char_table)
  emb = _final(char_e, pooled, W_out, b_out)
  return emb.reshape(32, 32, 2 * D)


# speedup vs baseline: 9.3994x; 1.0148x over previous
import functools

import jax
import jax.numpy as jnp
from jax import lax
from jax.experimental import pallas as pl
from jax.experimental.pallas import tpu as pltpu
from jax.experimental.pallas import tpu_sc as plsc

N_NODES = 10000
D = 128
N_EDGES = 320000
NUM_GRAPHS = 1024
NC = 2
NS = 16
NW = NC * NS
E_PER_W = N_EDGES // NW
CHUNK = 80
KIN = 5
KOUT = E_PER_W // (CHUNK * KIN)
ECH = 125
EKIN = 4
EKOUT = E_PER_W // (ECH * EKIN)
N_PAD = 10240
ROWS_PER_S = N_PAD // NS
G_PER_W = NUM_GRAPHS // NW
PCH = 64

_mesh = plsc.VectorSubcoreMesh(core_axis_name="c", subcore_axis_name="s")


def _zero_vmem(ref, nrows, width):
  def row(i, carry):
    for j in range(width // 16):
      ref[i, pl.ds(16 * j, 16)] = jnp.zeros((16,), jnp.float32)
    return carry
  lax.fori_loop(0, nrows, row, 0)


def _edge_body(feat, srcr, dstr, agg_out, src_a, dst_a, src_b, dst_b,
               rows_a, rows_b, agg_sh, sem_a, sem_b, sem_ia, sem_ib):
  cid = lax.axis_index("c")
  sid = lax.axis_index("s")
  wid = sid * NC + cid
  M = EKOUT // 2

  _zero_vmem(rows_a, 80, D)
  for c in range(ROWS_PER_S // 80):
    off = sid * ROWS_PER_S + c * 80
    pltpu.sync_copy(rows_a.at[pl.ds(0, 80)], agg_sh.at[pl.ds(off, 80)])
  plsc.subcore_barrier()

  pltpu.sync_copy(srcr.at[wid, 0], src_a)
  pltpu.sync_copy(dstr.at[wid, 0], dst_a)
  pltpu.async_copy(feat.at[src_a.at[0]], rows_a, sem_a)

  def outer(m, carry):
    dib0 = pltpu.async_copy(srcr.at[wid, 2 * m + 1], src_b, sem_ib)
    dib1 = pltpu.async_copy(dstr.at[wid, 2 * m + 1], dst_b, sem_ib)
    for j in range(EKIN):
      cur, csem = (rows_a, sem_a) if j % 2 == 0 else (rows_b, sem_b)
      nxt, nsem = (rows_b, sem_b) if j % 2 == 0 else (rows_a, sem_a)
      pltpu.make_async_copy(feat.at[src_a.at[j]], cur, csem).wait()
      if j + 1 < EKIN:
        pltpu.async_copy(feat.at[src_a.at[j + 1]], nxt, nsem)
      else:
        dib0.wait()
        dib1.wait()
        pltpu.async_copy(feat.at[src_b.at[0]], nxt, nsem)
      pltpu.sync_copy(cur, agg_sh.at[dst_a.at[j]], add=True)

    @pl.when(m < M - 1)
    def _():
      pltpu.async_copy(srcr.at[wid, 2 * m + 2], src_a, sem_ia)
      pltpu.async_copy(dstr.at[wid, 2 * m + 2], dst_a, sem_ia)

    for j in range(EKIN):
      cur, csem = (rows_a, sem_a) if j % 2 == 0 else (rows_b, sem_b)
      nxt, nsem = (rows_b, sem_b) if j % 2 == 0 else (rows_a, sem_a)
      pltpu.make_async_copy(feat.at[src_b.at[j]], cur, csem).wait()
      if j + 1 < EKIN:
        pltpu.async_copy(feat.at[src_b.at[j + 1]], nxt, nsem)
      else:
        @pl.when(m < M - 1)
        def _():
          pltpu.make_async_copy(srcr.at[wid, 0], src_a, sem_ia).wait()
          pltpu.make_async_copy(dstr.at[wid, 0], dst_a, sem_ia).wait()
          pltpu.async_copy(feat.at[src_a.at[0]], nxt, nsem)
      pltpu.sync_copy(cur, agg_sh.at[dst_b.at[j]], add=True)
    return carry
  lax.fori_loop(0, M, outer, 0)

  plsc.subcore_barrier()
  for c in range(ROWS_PER_S // 80):
    off = sid * ROWS_PER_S + c * 80
    pltpu.sync_copy(agg_sh.at[pl.ds(off, 80)],
                    agg_out.at[cid, pl.ds(off, 80)])


_edge_kernel = functools.partial(
    pl.kernel,
    out_type=jax.ShapeDtypeStruct((NC, N_PAD, D), jnp.float32),
    mesh=_mesh,
    scratch_types=[
        pltpu.VMEM((EKIN, ECH), jnp.int32),
        pltpu.VMEM((EKIN, ECH), jnp.int32),
        pltpu.VMEM((EKIN, ECH), jnp.int32),
        pltpu.VMEM((EKIN, ECH), jnp.int32),
        pltpu.VMEM((ECH, D), jnp.float32),
        pltpu.VMEM((ECH, D), jnp.float32),
        pltpu.VMEM_SHARED((N_PAD, D), jnp.float32),
        pltpu.SemaphoreType.DMA,
        pltpu.SemaphoreType.DMA,
        pltpu.SemaphoreType.DMA,
        pltpu.SemaphoreType.DMA,
    ],
    compiler_params=pltpu.CompilerParams(needs_layout_passes=False),
)(_edge_body)


def _deg_body(dstr, deg_out, dst_v, hist_v, sem):
  cid = lax.axis_index("c")
  sid = lax.axis_index("s")
  wid = sid * NC + cid

  _zero_vmem(hist_v, N_PAD // D, D)
  ones16 = jnp.full((16,), 1.0, jnp.float32)

  def outer(k, carry):
    pltpu.sync_copy(dstr.at[wid, k], dst_v)
    for j in range(KIN):
      for t in range(CHUNK // 16):
        idx = dst_v[j, pl.ds(16 * t, 16)]
        plsc.addupdate_scatter(hist_v, [idx >> 7, idx & 127], ones16)
    return carry
  lax.fori_loop(0, KOUT, outer, 0)

  pltpu.sync_copy(hist_v, deg_out.at[wid])


_deg_kernel = functools.partial(
    pl.kernel,
    out_type=jax.ShapeDtypeStruct((NW, N_PAD // D, D), jnp.float32),
    mesh=_mesh,
    scratch_types=[
        pltpu.VMEM((KIN, CHUNK), jnp.int32),
        pltpu.VMEM((N_PAD // D, D), jnp.float32),
        pltpu.SemaphoreType.DMA,
    ],
    compiler_params=pltpu.CompilerParams(needs_layout_passes=False),
)(_deg_body)


def _pool_body(h2r, nbp, idsr, ctab, pool_out, char_out,
               pool_v, rows_v, nbc_v, probe_v, cidx_v, crow_v, sem):
  cid = lax.axis_index("c")
  sid = lax.axis_index("s")
  wid = sid * NC + cid
  g0 = wid * G_PER_W

  pltpu.sync_copy(idsr, cidx_v)
  pltpu.async_copy(ctab.at[cidx_v.at[pl.ds(g0, G_PER_W)]], crow_v, sem).wait()
  pltpu.sync_copy(crow_v, char_out.at[pl.ds(g0, G_PER_W)])

  _zero_vmem(pool_v, G_PER_W, D)

  def search(target):
    def body(_, s):
      lo, hi = s
      mid = (lo + hi) // 2
      pltpu.sync_copy(nbp.at[pl.ds(mid * 8, 16)], probe_v.at[pl.ds(0, 16)])
      v = probe_v[pl.ds(0, 16)][0]
      lo2 = jnp.where(v < target, mid + 1, lo)
      hi2 = jnp.where(v < target, hi, mid)
      return (lo2, hi2)
    p, _ = lax.fori_loop(0, 11, body, (0, N_NODES // 8))
    base = jnp.maximum(p - 1, 0) * 8
    pltpu.sync_copy(nbp.at[pl.ds(base, 16)], probe_v.at[pl.ds(0, 16)])
    cnt = jnp.int32(0)
    for i in range(8):
      vi = probe_v[pl.ds(i, 16)][0]
      cnt = cnt + jnp.where(vi < target, 1, 0)
    return base + jnp.where(p > 0, cnt, 0)

  lo = search(g0)
  hi = search(g0 + G_PER_W)

  la = (lo // 8) * 8
  nchunks = (hi - la + PCH - 1) // PCH

  def ck(k, carry):
    n0 = la + k * PCH
    n0c = jnp.minimum(n0, N_NODES - PCH)
    pltpu.sync_copy(h2r.at[pl.ds(n0c, PCH)], rows_v)
    pltpu.sync_copy(nbp.at[pl.ds(n0c, PCH)], nbc_v.at[pl.ds(0, PCH)])

    def rb(r, c2):
      n = n0c + r
      ok = jnp.logical_and(n >= lo, n < hi)
      @pl.when(ok)
      def _():
        tgt = nbc_v[pl.ds(r, 16)][0] - g0
        for j in range(D // 16):
          sl = pl.ds(16 * j, 16)
          pool_v[tgt, sl] = jnp.maximum(pool_v[tgt, sl], rows_v[r, sl])
      return c2
    lax.fori_loop(0, PCH, rb, 0)
    return carry
  lax.fori_loop(0, nchunks, ck, 0)

  pltpu.sync_copy(pool_v, pool_out.at[pl.ds(g0, G_PER_W)])


_pool_kernel = functools.partial(
    pl.kernel,
    out_type=(
        jax.ShapeDtypeStruct((NUM_GRAPHS, D), jnp.float32),
        jax.ShapeDtypeStruct((NUM_GRAPHS, D), jnp.float32),
    ),
    mesh=_mesh,
    scratch_types=[
        pltpu.VMEM((G_PER_W, D), jnp.float32),
        pltpu.VMEM((PCH, D), jnp.float32),
        pltpu.VMEM((PCH + 16,), jnp.int32),
        pltpu.VMEM((32,), jnp.int32),
        pltpu.VMEM((NUM_GRAPHS,), jnp.int32),
        pltpu.VMEM((G_PER_W, D), jnp.float32),
        pltpu.SemaphoreType.DMA,
    ],
    compiler_params=pltpu.CompilerParams(needs_layout_passes=False),
)(_pool_body)


def _dense_body(agg_ref, deg_ref, x_ref, wl_ref, b_ref, wr_ref, o_ref):
  agg = agg_ref[0] + agg_ref[1]
  bn = agg.shape[0]
  hist = jnp.sum(deg_ref[...], axis=0)
  hrep = jnp.broadcast_to(hist[:, None, :], (bn // D, D, D)).reshape(bn, D)
  rows = lax.broadcasted_iota(jnp.int32, (bn, D), 0)
  cols = lax.broadcasted_iota(jnp.int32, (bn, D), 1)
  deg = jnp.sum(jnp.where((rows % D) == cols, hrep, 0.0), axis=1,
                keepdims=True)
  mean = agg / jnp.maximum(deg, 1.0)
  h = lax.dot_general(mean, wl_ref[...], (((1,), (1,)), ((), ())),
                      preferred_element_type=jnp.float32)
  h = h + b_ref[...][None, :]
  h = h + lax.dot_general(x_ref[...], wr_ref[...], (((1,), (1,)), ((), ())),
                          preferred_element_type=jnp.float32)
  o_ref[...] = jnp.maximum(h, 0.0)


def _dense(agg, deg, x, W_l, b_l, W_r):
  bn = 1024
  grid = (N_PAD // bn,)
  return pl.pallas_call(
      _dense_body,
      grid=grid,
      in_specs=[
          pl.BlockSpec((NC, bn, D), lambda i: (0, i, 0)),
          pl.BlockSpec((NW, bn // D, D), lambda i: (0, i, 0)),
          pl.BlockSpec((bn, D), lambda i: (i, 0)),
          pl.BlockSpec((D, D), lambda i: (0, 0)),
          pl.BlockSpec((D,), lambda i: (0,)),
          pl.BlockSpec((D, D), lambda i: (0, 0)),
      ],
      out_specs=pl.BlockSpec((bn, D), lambda i: (i, 0)),
      out_shape=jax.ShapeDtypeStruct((N_PAD, D), jnp.float32),
  )(agg, deg, x, W_l, b_l, W_r)


def _final_body(ch_ref, pool_ref, w_ref, b_ref, o_ref):
  o_ref[:, 0:D] = ch_ref[...]
  gf = lax.dot_general(pool_ref[...], w_ref[...], (((1,), (1,)), ((), ())),
                       preferred_element_type=jnp.float32)
  o_ref[:, D:2 * D] = gf + b_ref[...][None, :]


def _final(char_e, pooled, W_out, b_out):
  return pl.pallas_call(
      _final_body,
      out_shape=jax.ShapeDtypeStruct((NUM_GRAPHS, 2 * D), jnp.float32),
  )(char_e, pooled, W_out, b_out)


def kernel(data, x, edge_index, node_batch, char_table,
           W_l1, b_l1, W_r1, W_l2, b_l2, W_r2, W_out, b_out):
  ids = data.reshape(NUM_GRAPHS)
  srcr = edge_index[0].reshape(NW, EKOUT, EKIN, ECH)
  dstr = edge_index[1].reshape(NW, EKOUT, EKIN, ECH)
  dstr_deg = edge_index[1].reshape(NW, KOUT, KIN, CHUNK)
  x_pad = jnp.pad(x, ((0, N_PAD - N_NODES), (0, 0)))
  nbp = jnp.pad(node_batch, (0, 16), constant_values=NUM_GRAPHS)

  agg1 = _edge_kernel(x_pad, srcr, dstr)
  deg = _deg_kernel(dstr_deg)
  h1 = _dense(agg1, deg, x_pad, W_l1, b_l1, W_r1)
  agg2 = _edge_kernel(h1, srcr, dstr)
  h2 = _dense(agg2, deg, h1, W_l2, b_l2, W_r2)

  pooled, char_e = _pool_kernel(h2, nbp, ids, char_table)
  emb = _final(char_e, pooled, W_out, b_out)
  return emb.reshape(32, 32, 2 * D)
